# Initial kernel scaffold; baseline (speedup 1.0000x reference)
import jax
import jax.numpy as jnp
from jax.experimental import pallas as pl


def kernel(hidden_states, W, b, codevectors):
    raise NotImplementedError("write your pallas kernel here")


# speedup vs baseline: 1.7067x; 1.7067x over previous
import functools

import jax
import jax.numpy as jnp
from jax import lax
from jax.experimental import pallas as pl
from jax.experimental.pallas import tpu as pltpu
from jax.experimental.pallas import tpu_sc as plsc

G = 2
V = 320
GV = G * V
D = 128
H = 512
TOKENS = 4 * 2048
TILE = 1024
NT = TOKENS // TILE


def _proj_argmax_body(hs_ref, w_ref, b_ref, idx_ref, plx_ref, counts_ref):
    t = pl.program_id(0)

    @pl.when(t == 0)
    def _init():
        counts_ref[...] = jnp.zeros_like(counts_ref)

    logits = lax.dot_general(
        hs_ref[...], w_ref[...],
        dimension_numbers=(((1,), (1,)), ((), ())),
        preferred_element_type=jnp.float32,
    ) + b_ref[...]

    iota_v = lax.broadcasted_iota(jnp.int32, (TILE, V), 1)
    l0 = logits[:, :V]
    l1 = logits[:, V:]
    m0 = jnp.max(l0, axis=-1, keepdims=True)
    m1 = jnp.max(l1, axis=-1, keepdims=True)
    idx0 = jnp.min(jnp.where(l0 == m0, iota_v, V), axis=-1, keepdims=True)
    idx1 = jnp.min(jnp.where(l1 == m1, iota_v, V), axis=-1, keepdims=True)

    oh0 = (iota_v == idx0).astype(jnp.float32)
    oh1 = (iota_v == idx1).astype(jnp.float32)
    inc0 = jnp.sum(oh0, axis=0, keepdims=True)
    inc1 = jnp.sum(oh1, axis=0, keepdims=True)
    counts_ref[...] += jnp.concatenate([inc0, inc1], axis=0)

    idx_ref[...] = jnp.concatenate([idx0, idx1 + V], axis=1)

    @pl.when(t == NT - 1)
    def _finish():
        p = counts_ref[...] * (1.0 / TOKENS)
        ent = jnp.sum(p * jnp.log(p + 1e-7), axis=-1, keepdims=True)
        plx_ref[...] = jnp.sum(jnp.exp(-ent), axis=0, keepdims=True)


def _proj_argmax(hs, w, b2):
    return pl.pallas_call(
        _proj_argmax_body,
        grid=(NT,),
        in_specs=[
            pl.BlockSpec((TILE, H), lambda t: (t, 0)),
            pl.BlockSpec((GV, H), lambda t: (0, 0)),
            pl.BlockSpec((1, GV), lambda t: (0, 0)),
        ],
        out_specs=[
            pl.BlockSpec((TILE, G), lambda t: (t, 0)),
            pl.BlockSpec((1, 1), lambda t: (0, 0)),
        ],
        out_shape=[
            jax.ShapeDtypeStruct((TOKENS, G), jnp.int32),
            jax.ShapeDtypeStruct((1, 1), jnp.float32),
        ],
        scratch_shapes=[pltpu.VMEM((G, V), jnp.float32)],
    )(hs, w, b2)


_NC = 2
_NS = 16
_NW = _NC * _NS
_ROWS = TOKENS * G
_BPW = _ROWS // _NW


def _sc_gather_body(table_hbm, idx_hbm, out_hbm, idx_v, rows_v, sem):
    wid = lax.axis_index("s") * _NC + lax.axis_index("c")
    base = wid * _BPW
    pltpu.sync_copy(idx_hbm.at[pl.ds(base, _BPW)], idx_v)
    pltpu.async_copy(table_hbm.at[idx_v], rows_v, sem).wait()
    pltpu.sync_copy(rows_v, out_hbm.at[pl.ds(base, _BPW)])


@functools.partial(jax.jit, static_argnums=())
def _sc_gather(table, fidx):
    mesh = plsc.VectorSubcoreMesh(core_axis_name="c", subcore_axis_name="s")
    run = pl.kernel(
        _sc_gather_body,
        mesh=mesh,
        out_type=jax.ShapeDtypeStruct((_ROWS, D), jnp.float32),
        scratch_types=[
            pltpu.VMEM((_BPW,), jnp.int32),
            pltpu.VMEM((_BPW, D), jnp.float32),
            pltpu.SemaphoreType.DMA,
        ],
    )
    return run(table, fidx)


def kernel(hidden_states, W, b, codevectors):
    bsz, seq, hid = hidden_states.shape
    hs = hidden_states.reshape(bsz * seq, hid)
    b2 = b.reshape(1, GV)
    idx, plx = _proj_argmax(hs, W, b2)
    table = codevectors.reshape(GV, D)
    rows = _sc_gather(table, idx.reshape(_ROWS))
    codevecs = rows.reshape(bsz, seq, G * D)
    return codevecs, plx[0, 0]
